# baseline (speedup 1.0000x reference)
import functools

import jax
import jax.numpy as jnp
from jax import lax
from jax.experimental import pallas as pl
from jax.experimental.pallas import tpu as pltpu
from jax.experimental.pallas import tpu_sc as plsc

BATCH = 16384
EMBED = 64
HID = 32

NC = 2
NS = 16
NW = NC * NS
ROWS_PER_W = BATCH // NW
CHUNK = 128
NCHUNK = ROWS_PER_W // CHUNK


def _sc_gather(user_table, anime_table, uidx, aidx):
    mesh = plsc.VectorSubcoreMesh(core_axis_name="c", subcore_axis_name="s")

    @functools.partial(
        pl.kernel,
        mesh=mesh,
        compiler_params=pltpu.CompilerParams(use_tc_tiling_on_sc=False),
        out_type=[
            jax.ShapeDtypeStruct((BATCH, EMBED), jnp.float32),
            jax.ShapeDtypeStruct((BATCH, EMBED), jnp.float32),
        ],
        scratch_types=[
            pltpu.VMEM((NCHUNK, CHUNK), jnp.int32),
            pltpu.VMEM((NCHUNK, CHUNK), jnp.int32),
            pltpu.VMEM((ROWS_PER_W, EMBED), jnp.float32),
            pltpu.VMEM((ROWS_PER_W, EMBED), jnp.float32),
            pltpu.SemaphoreType.DMA,
        ],
    )
    def gather_kernel(ut_hbm, at_hbm, uid_hbm, aid_hbm, uout_hbm, aout_hbm,
                      uidx_v, aidx_v, urows_v, arows_v, sem):
        wid = lax.axis_index("s") * NC + lax.axis_index("c")
        base = wid * ROWS_PER_W
        pltpu.sync_copy(uid_hbm.at[wid], uidx_v)
        pltpu.sync_copy(aid_hbm.at[wid], aidx_v)
        copies = []
        for c in range(NCHUNK):
            copies.append(pltpu.async_copy(
                ut_hbm.at[uidx_v.at[c]],
                urows_v.at[pl.ds(c * CHUNK, CHUNK)], sem))
            copies.append(pltpu.async_copy(
                at_hbm.at[aidx_v.at[c]],
                arows_v.at[pl.ds(c * CHUNK, CHUNK)], sem))
        for cp in copies:
            cp.wait()
        pltpu.sync_copy(urows_v, uout_hbm.at[pl.ds(base, ROWS_PER_W)])
        pltpu.sync_copy(arows_v, aout_hbm.at[pl.ds(base, ROWS_PER_W)])

    return gather_kernel(user_table, anime_table, uidx, aidx)


def _mlp_body(ue_ref, ae_ref, w1u_ref, b1u_ref, w2u_ref, b2u_ref,
              w1a_ref, b1a_ref, w2a_ref, b2a_ref, out_ref):
    u = jnp.dot(ue_ref[...], w1u_ref[...],
                preferred_element_type=jnp.float32) + b1u_ref[...]
    u = jnp.maximum(u, 0.0)
    u = jnp.dot(u, w2u_ref[...],
                preferred_element_type=jnp.float32) + b2u_ref[...]
    u = jnp.maximum(u, 0.0)
    a = jnp.dot(ae_ref[...], w1a_ref[...],
                preferred_element_type=jnp.float32) + b1a_ref[...]
    a = jnp.maximum(a, 0.0)
    a = jnp.dot(a, w2a_ref[...],
                preferred_element_type=jnp.float32) + b2a_ref[...]
    a = jnp.maximum(a, 0.0)
    out_ref[...] = jnp.sum(u * a, axis=1)


def _tc_mlp(ue, ae, W1u, b1u, W2u, b2u, W1a, b1a, W2a, b2a):
    BLK = 2048
    grid = BATCH // BLK
    wspec = pl.BlockSpec((EMBED, HID), lambda i: (0, 0))
    w2spec = pl.BlockSpec((HID, HID), lambda i: (0, 0))
    bspec = pl.BlockSpec((1, HID), lambda i: (0, 0))
    espec = pl.BlockSpec((BLK, EMBED), lambda i: (i, 0))
    return pl.pallas_call(
        _mlp_body,
        grid=(grid,),
        in_specs=[espec, espec,
                  wspec, bspec, w2spec, bspec,
                  wspec, bspec, w2spec, bspec],
        out_specs=pl.BlockSpec((BLK,), lambda i: (i,)),
        out_shape=jax.ShapeDtypeStruct((BATCH,), jnp.float32),
    )(ue, ae,
      W1u.T, b1u.reshape(1, HID), W2u.T, b2u.reshape(1, HID),
      W1a.T, b1a.reshape(1, HID), W2a.T, b2a.reshape(1, HID))


def kernel(user_ids, anime_ids, user_table, anime_table,
           W1u, b1u, W2u, b2u, W1a, b1a, W2a, b2a):
    uidx = user_ids.astype(jnp.int32).reshape(NW, NCHUNK, CHUNK)
    aidx = anime_ids.astype(jnp.int32).reshape(NW, NCHUNK, CHUNK)
    ue, ae = _sc_gather(user_table, anime_table, uidx, aidx)
    return _tc_mlp(ue, ae, W1u, b1u, W2u, b2u, W1a, b1a, W2a, b2a)
